# Initial kernel scaffold; baseline (speedup 1.0000x reference)
import jax
import jax.numpy as jnp
from jax.experimental import pallas as pl


def kernel(x_LL, x_LH, x_HL, x_HH):
    raise NotImplementedError("write your pallas kernel here")


# speedup vs baseline: 9.6084x; 9.6084x over previous
import numpy as np
import jax
import jax.numpy as jnp
from jax.experimental import pallas as pl

_H = 96
_W = 96


def _make_scatter_matrix() -> np.ndarray:
    signs = {
        0: (1, 1, 1, 1),
        1: (-1, 1, -1, 1),
        2: (-1, -1, 1, 1),
        3: (1, -1, -1, 1),
    }
    m = np.zeros((4 * _W, 4 * _W), np.float32)
    w = np.arange(_W)
    for k in range(4):
        c1, c2, c3, c4 = signs[k]
        r = k * _W + w
        m[r, 2 * w] = 0.5 * c1
        m[r, 2 * w + 1] = 0.5 * c3
        m[r, 2 * _W + 2 * w] = 0.5 * c2
        m[r, 2 * _W + 2 * w + 1] = 0.5 * c4
    return m


def _iwt_block(ll_ref, lh_ref, hl_ref, hh_ref, m_ref, out_ref):
    f32 = jnp.float32
    bf16 = jnp.bfloat16
    acc = jnp.dot(ll_ref[...].astype(bf16), m_ref[0:_W],
                  preferred_element_type=f32)
    acc += jnp.dot(lh_ref[...].astype(bf16), m_ref[_W:2 * _W],
                   preferred_element_type=f32)
    acc += jnp.dot(hl_ref[...].astype(bf16), m_ref[2 * _W:3 * _W],
                   preferred_element_type=f32)
    acc += jnp.dot(hh_ref[...].astype(bf16), m_ref[3 * _W:4 * _W],
                   preferred_element_type=f32)
    out_ref[...] = acc


def kernel(x_LL, x_LH, x_HL, x_HH):
    b, c, d, h, w = x_LL.shape
    rows = b * c * d * h
    blk = 1024
    m = jnp.asarray(_make_scatter_matrix(), dtype=jnp.bfloat16)
    flat = lambda x: x.reshape(rows, w)
    out = pl.pallas_call(
        _iwt_block,
        grid=(rows // blk,),
        in_specs=[pl.BlockSpec((blk, w), lambda i: (i, 0))] * 4
        + [pl.BlockSpec((4 * w, 4 * w), lambda i: (0, 0))],
        out_specs=pl.BlockSpec((blk, 4 * w), lambda i: (i, 0)),
        out_shape=jax.ShapeDtypeStruct((rows, 4 * w), x_LL.dtype),
    )(flat(x_LL), flat(x_LH), flat(x_HL), flat(x_HH), m)
    return out.reshape(b, c, d, 2 * h, 2 * w)


# speedup vs baseline: 10.9661x; 1.1413x over previous
import numpy as np
import jax
import jax.numpy as jnp
from jax.experimental import pallas as pl

_H = 96
_W = 96


def _make_scatter_matrix() -> np.ndarray:
    signs = {
        0: (1, 1, 1, 1),
        1: (-1, 1, -1, 1),
        2: (-1, -1, 1, 1),
        3: (1, -1, -1, 1),
    }
    m = np.zeros((4 * _W, 4 * _W), np.float32)
    w = np.arange(_W)
    for k in range(4):
        c1, c2, c3, c4 = signs[k]
        r = k * _W + w
        m[r, 2 * w] = 0.5 * c1
        m[r, 2 * w + 1] = 0.5 * c3
        m[r, 2 * _W + 2 * w] = 0.5 * c2
        m[r, 2 * _W + 2 * w + 1] = 0.5 * c4
    return m


def _iwt_block(ll_ref, lh_ref, hl_ref, hh_ref, m_ref, out_ref):
    f32 = jnp.float32
    bf16 = jnp.bfloat16
    nb = ll_ref.shape[0]
    rows = nb * _H

    def rows2d(ref):
        return ref[...].reshape(rows, _W).astype(bf16)

    ll = rows2d(ll_ref)
    lh = rows2d(lh_ref)
    hl = rows2d(hl_ref)
    hh = rows2d(hh_ref)

    def comb(cols):
        acc = jnp.dot(ll, m_ref[0:_W, cols], preferred_element_type=f32)
        acc += jnp.dot(lh, m_ref[_W:2 * _W, cols], preferred_element_type=f32)
        acc += jnp.dot(hl, m_ref[2 * _W:3 * _W, cols], preferred_element_type=f32)
        acc += jnp.dot(hh, m_ref[3 * _W:4 * _W, cols], preferred_element_type=f32)
        return acc.reshape(nb, _H, 2 * _W)

    even = comb(slice(0, 2 * _W))
    odd = comb(slice(2 * _W, 4 * _W))
    out_ref[...] = jnp.stack([even, odd], axis=2).reshape(nb, 2 * _H, 2 * _W)


def kernel(x_LL, x_LH, x_HL, x_HH):
    b, c, d, h, w = x_LL.shape
    n = b * c * d
    nb = 8
    m = jnp.asarray(_make_scatter_matrix(), dtype=jnp.bfloat16)
    flat = lambda x: x.reshape(n, h, w)
    out = pl.pallas_call(
        _iwt_block,
        grid=(n // nb,),
        in_specs=[pl.BlockSpec((nb, h, w), lambda i: (i, 0, 0))] * 4
        + [pl.BlockSpec((4 * w, 4 * w), lambda i: (0, 0))],
        out_specs=pl.BlockSpec((nb, 2 * h, 2 * w), lambda i: (i, 0, 0)),
        out_shape=jax.ShapeDtypeStruct((n, 2 * h, 2 * w), x_LL.dtype),
    )(flat(x_LL), flat(x_LH), flat(x_HL), flat(x_HH), m)
    return out.reshape(b, c, d, 2 * h, 2 * w)


# speedup vs baseline: 14.2617x; 1.3005x over previous
import numpy as np
import jax
import jax.numpy as jnp
from jax.experimental import pallas as pl

_H = 96
_W = 96


def _make_scatter_matrix() -> np.ndarray:
    signs = {
        0: (1, 1, 1, 1),
        1: (-1, 1, -1, 1),
        2: (-1, -1, 1, 1),
        3: (1, -1, -1, 1),
    }
    m = np.zeros((4 * _W, 4 * _W), np.float32)
    w = np.arange(_W)
    for k in range(4):
        c1, c2, c3, c4 = signs[k]
        r = k * _W + w
        m[r, 2 * w] = 0.5 * c1
        m[r, 2 * w + 1] = 0.5 * c3
        m[r, 2 * _W + 2 * w] = 0.5 * c2
        m[r, 2 * _W + 2 * w + 1] = 0.5 * c4
    return m


def _make_row_scatter() -> np.ndarray:
    l = np.zeros((2, 2 * _H, _H), np.float32)
    h = np.arange(_H)
    l[0, 2 * h, h] = 1.0
    l[1, 2 * h + 1, h] = 1.0
    return l


def _iwt_block(ll_ref, lh_ref, hl_ref, hh_ref, m_ref, l_ref, out_ref):
    f32 = jnp.float32
    bf16 = jnp.bfloat16
    nb = ll_ref.shape[0]
    rows = nb * _H

    def rows2d(ref):
        return ref[...].reshape(rows, _W).astype(bf16)

    ll = rows2d(ll_ref)
    lh = rows2d(lh_ref)
    hl = rows2d(hl_ref)
    hh = rows2d(hh_ref)

    def comb(cols):
        acc = jnp.dot(ll, m_ref[0:_W, cols], preferred_element_type=f32)
        acc += jnp.dot(lh, m_ref[_W:2 * _W, cols], preferred_element_type=f32)
        acc += jnp.dot(hl, m_ref[2 * _W:3 * _W, cols], preferred_element_type=f32)
        acc += jnp.dot(hh, m_ref[3 * _W:4 * _W, cols], preferred_element_type=f32)
        return acc.astype(bf16).reshape(nb, _H, 2 * _W)

    even = comb(slice(0, 2 * _W))
    odd = comb(slice(2 * _W, 4 * _W))
    le = jnp.broadcast_to(l_ref[0], (nb, 2 * _H, _H))
    lo = jnp.broadcast_to(l_ref[1], (nb, 2 * _H, _H))
    dn = (((2,), (1,)), ((0,), (0,)))
    out = jax.lax.dot_general(le, even, dn, preferred_element_type=f32)
    out += jax.lax.dot_general(lo, odd, dn, preferred_element_type=f32)
    out_ref[...] = out


def kernel(x_LL, x_LH, x_HL, x_HH):
    b, c, d, h, w = x_LL.shape
    n = b * c * d
    nb = 8
    m = jnp.asarray(_make_scatter_matrix(), dtype=jnp.bfloat16)
    l = jnp.asarray(_make_row_scatter(), dtype=jnp.bfloat16)
    flat = lambda x: x.reshape(n, h, w)
    out = pl.pallas_call(
        _iwt_block,
        grid=(n // nb,),
        in_specs=[pl.BlockSpec((nb, h, w), lambda i: (i, 0, 0))] * 4
        + [pl.BlockSpec((4 * w, 4 * w), lambda i: (0, 0)),
           pl.BlockSpec((2, 2 * h, h), lambda i: (0, 0, 0))],
        out_specs=pl.BlockSpec((nb, 2 * h, 2 * w), lambda i: (i, 0, 0)),
        out_shape=jax.ShapeDtypeStruct((n, 2 * h, 2 * w), x_LL.dtype),
    )(flat(x_LL), flat(x_LH), flat(x_HL), flat(x_HH), m, l)
    return out.reshape(b, c, d, 2 * h, 2 * w)


# speedup vs baseline: 15.2616x; 1.0701x over previous
from kernel_sc import sc_iwt


def kernel(x_LL, x_LH, x_HL, x_HH):
    return sc_iwt(x_LL, x_LH, x_HL, x_HH)


# speedup vs baseline: 15.5193x; 1.0169x over previous
import dataclasses
import jax
import jax.numpy as jnp
from jax.experimental import pallas as pl
from jax.experimental.pallas import tpu as pltpu
from jax.experimental.pallas import tpu_sc as plsc


def _sc_iwt(x_LL, x_LH, x_HL, x_HH):
    n, h, w = x_LL.shape
    hh_blk = h // 2
    nchunk = w // 16

    mesh = plsc.VectorSubcoreMesh(core_axis_name="core",
                                  subcore_axis_name="subcore")
    cp = pltpu.CompilerParams()
    if "needs_layout_passes" in pltpu.CompilerParams.__dataclass_fields__:
        cp = dataclasses.replace(cp, needs_layout_passes=False)

    @pl.kernel(out_type=jax.ShapeDtypeStruct((n, 2 * h, 2 * w), x_LL.dtype),
               mesh=mesh, compiler_params=cp)
    def sc_kernel(ll_hbm, lh_hbm, hl_hbm, hh_hbm, o_hbm):
        def body(ll_ref, lh_ref, hl_ref, hh_ref, out_ref):
            lane2 = jax.lax.iota(jnp.int32, 16) * 2
            zero16 = jnp.zeros((16,), jnp.int32)

            @pl.loop(0, hh_blk)
            def _(hi):
                re = zero16 + 2 * hi
                ro = re + 1
                for ci in range(nchunk):
                    sl = (0, hi, pl.ds(ci * 16, 16))
                    ll = ll_ref[sl]
                    lh = lh_ref[sl]
                    hl = hl_ref[sl]
                    hh = hh_ref[sl]
                    p = (ll - hl) * 0.5
                    q = (lh - hh) * 0.5
                    r = (ll + hl) * 0.5
                    s = (lh + hh) * 0.5
                    ce = lane2 + 32 * ci
                    co = ce + 1
                    plsc.store_scatter(out_ref, [zero16, re, ce], p - q)
                    plsc.store_scatter(out_ref, [zero16, re, co], r - s)
                    plsc.store_scatter(out_ref, [zero16, ro, ce], p + q)
                    plsc.store_scatter(out_ref, [zero16, ro, co], r + s)

        pltpu.emit_pipeline(
            body,
            grid=(n, 2),
            in_specs=[pl.BlockSpec((1, hh_blk, w), lambda i, j: (i, j, 0))] * 4,
            out_specs=[pl.BlockSpec((1, h, 2 * w), lambda i, j: (i, j, 0))],
            core_axis_name=("core", "subcore"),
            dimension_semantics=(pltpu.PARALLEL, pltpu.PARALLEL),
        )(ll_hbm, lh_hbm, hl_hbm, hh_hbm, o_hbm)

    return sc_kernel(x_LL, x_LH, x_HL, x_HH)


def kernel(x_LL, x_LH, x_HL, x_HH):
    b, c, d, h, w = x_LL.shape
    n = b * c * d
    flat = lambda x: x.reshape(n, h, w)
    out = _sc_iwt(flat(x_LL), flat(x_LH), flat(x_HL), flat(x_HH))
    return out.reshape(b, c, d, 2 * h, 2 * w)


# speedup vs baseline: 21.0031x; 1.3533x over previous
import dataclasses
import jax
import jax.numpy as jnp
from jax.experimental import pallas as pl
from jax.experimental.pallas import tpu as pltpu
from jax.experimental.pallas import tpu_sc as plsc


def _sc_iwt(x_LL, x_LH, x_HL, x_HH):
    n, h, w = x_LL.shape
    hh_blk = h // 2
    nchunk = w // 16

    mesh = plsc.VectorSubcoreMesh(core_axis_name="core",
                                  subcore_axis_name="subcore")
    cp = pltpu.CompilerParams()
    if "needs_layout_passes" in pltpu.CompilerParams.__dataclass_fields__:
        cp = dataclasses.replace(cp, needs_layout_passes=False)

    @pl.kernel(out_type=jax.ShapeDtypeStruct((n, 2 * h, 2 * w), x_LL.dtype),
               mesh=mesh, compiler_params=cp)
    def sc_kernel(ll_hbm, lh_hbm, hl_hbm, hh_hbm, o_hbm):
        def body(ll_ref, lh_ref, hl_ref, hh_ref, out_ref):
            lane2 = jax.lax.iota(jnp.int32, 16) * 2
            zero16 = jnp.zeros((16,), jnp.int32)
            out2 = out_ref.at[0]

            @plsc.parallel_loop(0, hh_blk, 1, unroll=2)
            def _(hi):
                re = zero16 + 2 * hi
                ro = re + 1
                for ci in range(nchunk):
                    sl = (0, hi, pl.ds(ci * 16, 16))
                    ll = ll_ref[sl]
                    lh = lh_ref[sl]
                    hl = hl_ref[sl]
                    hh = hh_ref[sl]
                    p = (ll - hl) * 0.5
                    q = (lh - hh) * 0.5
                    r = (ll + hl) * 0.5
                    s = (lh + hh) * 0.5
                    ce = lane2 + 32 * ci
                    co = ce + 1
                    plsc.store_scatter(out2, [re, ce], p - q)
                    plsc.store_scatter(out2, [re, co], r - s)
                    plsc.store_scatter(out2, [ro, ce], p + q)
                    plsc.store_scatter(out2, [ro, co], r + s)

        pltpu.emit_pipeline(
            body,
            grid=(n, 2),
            in_specs=[pl.BlockSpec((1, hh_blk, w), lambda i, j: (i, j, 0))] * 4,
            out_specs=[pl.BlockSpec((1, h, 2 * w), lambda i, j: (i, j, 0))],
            core_axis_name=("core", "subcore"),
            dimension_semantics=(pltpu.PARALLEL, pltpu.PARALLEL),
        )(ll_hbm, lh_hbm, hl_hbm, hh_hbm, o_hbm)

    return sc_kernel(x_LL, x_LH, x_HL, x_HH)


def kernel(x_LL, x_LH, x_HL, x_HH):
    b, c, d, h, w = x_LL.shape
    n = b * c * d
    flat = lambda x: x.reshape(n, h, w)
    out = _sc_iwt(flat(x_LL), flat(x_LH), flat(x_HL), flat(x_HH))
    return out.reshape(b, c, d, 2 * h, 2 * w)


# speedup vs baseline: 21.1048x; 1.0048x over previous
import dataclasses
import jax
import jax.numpy as jnp
from jax.experimental import pallas as pl
from jax.experimental.pallas import tpu as pltpu
from jax.experimental.pallas import tpu_sc as plsc


def _sc_iwt(x_LL, x_LH, x_HL, x_HH):
    n, h, w = x_LL.shape
    hh_blk = h // 2
    nchunk = w // 16

    mesh = plsc.VectorSubcoreMesh(core_axis_name="core",
                                  subcore_axis_name="subcore")
    cp = dataclasses.replace(pltpu.CompilerParams(),
                             needs_layout_passes=False)

    @pl.kernel(out_type=jax.ShapeDtypeStruct((n, 2 * h, 2 * w), x_LL.dtype),
               mesh=mesh, compiler_params=cp)
    def sc_kernel(ll_hbm, lh_hbm, hl_hbm, hh_hbm, o_hbm):
        def body(ll_ref, lh_ref, hl_ref, hh_ref, out_ref):
            lane2 = jax.lax.iota(jnp.int32, 16) * 2
            zero16 = jnp.zeros((16,), jnp.int32)
            out2 = out_ref.at[0]

            @plsc.parallel_loop(0, hh_blk, 1, unroll=4)
            def _(hi):
                re = zero16 + 2 * hi
                ro = re + 1
                for ci in range(nchunk):
                    sl = (0, hi, pl.ds(ci * 16, 16))
                    ll = ll_ref[sl]
                    lh = lh_ref[sl]
                    hl = hl_ref[sl]
                    hh = hh_ref[sl]
                    p = (ll - hl) * 0.5
                    q = (lh - hh) * 0.5
                    r = (ll + hl) * 0.5
                    s = (lh + hh) * 0.5
                    ce = lane2 + 32 * ci
                    co = ce + 1
                    plsc.store_scatter(out2, [re, ce], p - q)
                    plsc.store_scatter(out2, [re, co], r - s)
                    plsc.store_scatter(out2, [ro, ce], p + q)
                    plsc.store_scatter(out2, [ro, co], r + s)

        pltpu.emit_pipeline(
            body,
            grid=(n, 2),
            in_specs=[pl.BlockSpec((1, hh_blk, w), lambda i, j: (i, j, 0))] * 4,
            out_specs=[pl.BlockSpec((1, h, 2 * w), lambda i, j: (i, j, 0))],
            core_axis_name=("core", "subcore"),
            dimension_semantics=(pltpu.PARALLEL, pltpu.PARALLEL),
        )(ll_hbm, lh_hbm, hl_hbm, hh_hbm, o_hbm)

    return sc_kernel(x_LL, x_LH, x_HL, x_HH)


def kernel(x_LL, x_LH, x_HL, x_HH):
    b, c, d, h, w = x_LL.shape
    n = b * c * d
    flat = lambda x: x.reshape(n, h, w)
    out = _sc_iwt(flat(x_LL), flat(x_LH), flat(x_HL), flat(x_HH))
    return out.reshape(b, c, d, 2 * h, 2 * w)
